# baseline (device time: 65477 ns/iter reference)
import jax
import jax.numpy as jnp
from jax import lax
from jax.experimental import pallas as pl
from jax.experimental.pallas import tpu as pltpu

N_DEV = 32


def kernel(x, router_W, route_idx, expert_W):
    T, D = x.shape
    _, N_EXP = router_W.shape
    E_per, _, H = expert_W.shape
    K = E_per * D

    def body(x_ref, rw_ref, idx_ref, ew_ref, out_ref,
             comm_ref, xg_ref, send_sems, recv_sems):
        my = lax.axis_index("i")
        left = lax.rem(my + N_DEV - 1, N_DEV)
        right = lax.rem(my + 1, N_DEV)

        barrier_sem = pltpu.get_barrier_semaphore()

        def bar_signal(d, carry):
            pl.semaphore_signal(
                barrier_sem, inc=1,
                device_id=(lax.rem(my + d, N_DEV),),
                device_id_type=pl.DeviceIdType.MESH,
            )
            return carry

        lax.fori_loop(1, N_DEV, bar_signal, 0)
        pl.semaphore_wait(barrier_sem, N_DEV - 1)

        xf = x_ref[...]
        scores = jnp.dot(xf, rw_ref[...],
                         preferred_element_type=jnp.float32)
        probs = jax.nn.softmax(scores, axis=-1)
        e_ids = lax.broadcasted_iota(jnp.int32, (T, N_EXP), 1)
        sel = (e_ids == idx_ref[:, 0:1]) | (e_ids == idx_ref[:, 1:2])
        w = jnp.where(sel, probs, 0.0)
        w = w / jnp.sum(w, axis=-1, keepdims=True)

        comm_ref[pl.ds(my * K, K)] = (
            ew_ref[...].astype(jnp.bfloat16).reshape(K, H)
        )

        def send_to(d, carry):
            dst = lax.rem(my + d, N_DEV)
            send = pltpu.make_async_remote_copy(
                src_ref=comm_ref.at[pl.ds(my * K, K)],
                dst_ref=comm_ref.at[pl.ds(my * K, K)],
                send_sem=send_sems.at[dst],
                recv_sem=recv_sems.at[my],
                device_id=(dst,),
                device_id_type=pl.DeviceIdType.MESH,
            )
            send.start()
            return carry

        lax.fori_loop(1, N_DEV, send_to, 0)

        for e in range(N_EXP):
            xg_ref[:, e * D:(e + 1) * D] = (
                xf * w[:, e:e + 1]
            ).astype(jnp.bfloat16)

        GROUP = 8
        for g in range(N_DEV // GROUP):
            for j in range(GROUP):
                o = g * GROUP + j

                @pl.when(o != my)
                def _(o=o):
                    recv = pltpu.make_async_remote_copy(
                        src_ref=comm_ref.at[pl.ds(o * K, K)],
                        dst_ref=comm_ref.at[pl.ds(o * K, K)],
                        send_sem=send_sems.at[o],
                        recv_sem=recv_sems.at[o],
                        device_id=(o,),
                        device_id_type=pl.DeviceIdType.MESH,
                    )
                    recv.wait_recv()

            lo, hi = g * GROUP * K, (g + 1) * GROUP * K
            slab = jnp.dot(xg_ref[:, lo:hi], comm_ref[lo:hi, :],
                           preferred_element_type=jnp.float32)
            if g == 0:
                out_ref[...] = slab
            else:
                out_ref[...] += slab

        def drain(d, carry):
            dst = lax.rem(my + d, N_DEV)
            send = pltpu.make_async_remote_copy(
                src_ref=comm_ref.at[pl.ds(my * K, K)],
                dst_ref=comm_ref.at[pl.ds(my * K, K)],
                send_sem=send_sems.at[dst],
                recv_sem=recv_sems.at[my],
                device_id=(dst,),
                device_id_type=pl.DeviceIdType.MESH,
            )
            send.wait_send()
            return carry

        lax.fori_loop(1, N_DEV, drain, 0)

    return pl.pallas_call(
        body,
        out_shape=jax.ShapeDtypeStruct((T, H), jnp.float32),
        in_specs=[pl.BlockSpec(memory_space=pltpu.VMEM)] * 4,
        out_specs=pl.BlockSpec(memory_space=pltpu.VMEM),
        scratch_shapes=[
            pltpu.VMEM((N_DEV * K, H), jnp.bfloat16),
            pltpu.VMEM((T, N_EXP * D), jnp.bfloat16),
            pltpu.SemaphoreType.DMA((N_DEV,)),
            pltpu.SemaphoreType.DMA((N_DEV,)),
        ],
        compiler_params=pltpu.CompilerParams(collective_id=0),
    )(x, router_W, route_idx, expert_W)
